# Initial kernel scaffold; baseline (speedup 1.0000x reference)
import jax
import jax.numpy as jnp
from jax.experimental import pallas as pl


def kernel(g, h, Q1_w, Q1_b, W1_w, W1_b, Q2_w, Q2_b, W2_w, W2_b, G_w, G_b, g_scale):
    raise NotImplementedError("write your pallas kernel here")


# speedup vs baseline: 5.1907x; 5.1907x over previous
import functools

import jax
import jax.numpy as jnp
from jax import lax
from jax.experimental import pallas as pl
from jax.experimental.pallas import tpu as pltpu
from jax.experimental.pallas import tpu_sc as plsc

N = 10000
E = 320000
D = 128
N_PAD = 10240
BLK = 512
NC = 2
NS = 16
NW = NC * NS
CHUNK = 80
L = 16


def _make_edge_scatter(with_deg):
    e_per_w = E // NW
    n_it = e_per_w // CHUNK
    rows_ps = N_PAD // NS
    mesh = plsc.VectorSubcoreMesh(core_axis_name="c", subcore_axis_name="s")

    out_type = [jax.ShapeDtypeStruct((NC * N_PAD, D), jnp.float32)]
    scratch = [
        pltpu.VMEM((CHUNK,), jnp.int32),
        pltpu.VMEM((CHUNK,), jnp.int32),
        pltpu.VMEM((CHUNK, D), jnp.float32),
        pltpu.VMEM_SHARED((N_PAD, D), jnp.float32),
        pltpu.SemaphoreType.DMA,
    ]
    if with_deg:
        out_type.append(jax.ShapeDtypeStruct((NW, N_PAD), jnp.float32))
        scratch.append(pltpu.VMEM((e_per_w,), jnp.int32))
        scratch.append(pltpu.VMEM((N_PAD,), jnp.float32))

    @functools.partial(pl.kernel, mesh=mesh, out_type=out_type,
                       scratch_types=scratch,
                       compiler_params=pltpu.CompilerParams(
                           needs_layout_passes=False))
    def edge_scatter(src_hbm, dst_hbm, m_hbm, zeros_hbm, *refs):
        if with_deg:
            (out_hbm, deg_hbm, src_v, dst_v, rows_v, agg_s, sem,
             dslab_v, hist_v) = refs
        else:
            out_hbm, src_v, dst_v, rows_v, agg_s, sem = refs
        cid = lax.axis_index("c")
        sid = lax.axis_index("s")
        wid = cid * NS + sid
        r0 = sid * rows_ps
        base = wid * e_per_w

        pltpu.sync_copy(zeros_hbm.at[pl.ds(r0, rows_ps)],
                        agg_s.at[pl.ds(r0, rows_ps)])

        if with_deg:
            zeros16 = jnp.zeros((L,), jnp.float32)

            def zbody(i, c):
                hist_v[pl.ds(i * L, L)] = zeros16
                return c

            lax.fori_loop(0, N_PAD // L, zbody, 0)
            pltpu.sync_copy(dst_hbm.at[pl.ds(base, e_per_w)], dslab_v)
            ones16 = jnp.full((L,), 1.0, jnp.float32)

            def hbody(i, c):
                idx16 = dslab_v[pl.ds(i * L, L)]
                plsc.addupdate_scatter(hist_v, [idx16], ones16)
                return c

            lax.fori_loop(0, e_per_w // L, hbody, 0)
            pltpu.sync_copy(hist_v, deg_hbm.at[wid])

        plsc.subcore_barrier()

        def body(i, carry):
            off = base + i * CHUNK
            pltpu.sync_copy(src_hbm.at[pl.ds(off, CHUNK)], src_v)
            pltpu.sync_copy(dst_hbm.at[pl.ds(off, CHUNK)], dst_v)
            pltpu.async_copy(m_hbm.at[src_v], rows_v, sem).wait()
            pltpu.sync_copy(rows_v, agg_s.at[dst_v], add=True)
            return carry

        lax.fori_loop(0, n_it, body, 0)
        plsc.subcore_barrier()

        pltpu.sync_copy(agg_s.at[pl.ds(r0, rows_ps)],
                        out_hbm.at[pl.ds(cid * N_PAD + r0, rows_ps)])

    return edge_scatter


_edge_scatter_l1 = _make_edge_scatter(True)
_edge_scatter_l2 = _make_edge_scatter(False)


def _pre1_body(h_ref, qt_ref, qb_ref, o_ref):
    m = jnp.dot(h_ref[...], qt_ref[...], preferred_element_type=jnp.float32)
    o_ref[...] = jnp.maximum(m + qb_ref[...], 0.0)


def _mid_body(h_ref, a0_ref, a1_ref, degp_ref, w1a_ref, w1b_ref, w1b_b_ref,
              q2t_ref, q2b_ref, h1_ref, m2_ref, deg_ref):
    deg = jnp.sum(degp_ref[...], axis=0)
    hn = (a0_ref[...] + a1_ref[...]) / jnp.maximum(deg, 1.0)[:, None]
    z = jnp.dot(h_ref[...], w1a_ref[...], preferred_element_type=jnp.float32)
    z = z + jnp.dot(hn, w1b_ref[...], preferred_element_type=jnp.float32)
    z = jnp.maximum(z + w1b_b_ref[...], 0.0)
    nrm = jnp.sqrt(jnp.sum(z * z, axis=1, keepdims=True))
    h1 = z / (nrm + 1e-6)
    h1_ref[...] = h1
    m2 = jnp.dot(h1, q2t_ref[...], preferred_element_type=jnp.float32)
    m2_ref[...] = jnp.maximum(m2 + q2b_ref[...], 0.0)
    deg_ref[...] = deg


def _post_body(h1_ref, a0_ref, a1_ref, deg_ref, w2a_ref, w2b_ref, w2b_b_ref,
               gt_ref, gb_ref, gs_ref, o_ref):
    deg = deg_ref[...]
    hn = (a0_ref[...] + a1_ref[...]) / jnp.maximum(deg, 1.0)[:, None]
    z = jnp.dot(h1_ref[...], w2a_ref[...], preferred_element_type=jnp.float32)
    z = z + jnp.dot(hn, w2b_ref[...], preferred_element_type=jnp.float32)
    z = jnp.maximum(z + w2b_b_ref[...], 0.0)
    nrm = jnp.sqrt(jnp.sum(z * z, axis=1, keepdims=True))
    h2 = z / (nrm + 1e-6)
    out = jnp.dot(h2, gt_ref[...], preferred_element_type=jnp.float32)
    o_ref[...] = gs_ref[...] * jnp.maximum(out + gb_ref[...], 0.0)


def _row_spec(width):
    return pl.BlockSpec((BLK, width), lambda i: (i, 0))


def _full_spec(shape):
    return pl.BlockSpec(shape, lambda i: tuple(0 for _ in shape))


_GRID = N_PAD // BLK
_vec_spec = pl.BlockSpec((BLK,), lambda i: (i,))

_pre1 = pl.pallas_call(
    _pre1_body,
    grid=(_GRID,),
    in_specs=[_row_spec(D), _full_spec((D, D)), _full_spec((1, D))],
    out_specs=_row_spec(D),
    out_shape=jax.ShapeDtypeStruct((N_PAD, D), jnp.float32),
)

_mid = pl.pallas_call(
    _mid_body,
    grid=(_GRID,),
    in_specs=[_row_spec(D), _row_spec(D), _row_spec(D),
              pl.BlockSpec((NW, BLK), lambda i: (0, i)),
              _full_spec((D, D)), _full_spec((D, D)), _full_spec((1, D)),
              _full_spec((D, D)), _full_spec((1, D))],
    out_specs=[_row_spec(D), _row_spec(D), _vec_spec],
    out_shape=[jax.ShapeDtypeStruct((N_PAD, D), jnp.float32),
               jax.ShapeDtypeStruct((N_PAD, D), jnp.float32),
               jax.ShapeDtypeStruct((N_PAD,), jnp.float32)],
)

_post = pl.pallas_call(
    _post_body,
    grid=(_GRID,),
    in_specs=[_row_spec(D), _row_spec(D), _row_spec(D), _vec_spec,
              _full_spec((D, D)), _full_spec((D, D)), _full_spec((1, D)),
              _full_spec((D, D)), _full_spec((1, D)), _full_spec((1, D))],
    out_specs=_row_spec(D),
    out_shape=jax.ShapeDtypeStruct((N_PAD, D), jnp.float32),
)


@jax.jit
def kernel(g, h, Q1_w, Q1_b, W1_w, W1_b, Q2_w, Q2_b, W2_w, W2_b, G_w, G_b,
           g_scale):
    src = g[0].astype(jnp.int32)
    dst = g[1].astype(jnp.int32)
    h_pad = jnp.zeros((N_PAD, D), jnp.float32).at[:N].set(h)
    zeros_nd = jnp.zeros((N_PAD, D), jnp.float32)

    m1 = _pre1(h_pad, Q1_w.T, Q1_b[None, :])
    agg1, degp = _edge_scatter_l1(src, dst, m1, zeros_nd)

    h1, m2, deg = _mid(h_pad, agg1[:N_PAD], agg1[N_PAD:], degp,
                       W1_w[:, :D].T, W1_w[:, D:].T, W1_b[None, :],
                       Q2_w.T, Q2_b[None, :])

    (agg2,) = _edge_scatter_l2(src, dst, m2, zeros_nd)

    gs = jnp.broadcast_to(g_scale.astype(jnp.float32), (1, D))
    out = _post(h1, agg2[:N_PAD], agg2[N_PAD:], deg,
                W2_w[:, :D].T, W2_w[:, D:].T, W2_b[None, :],
                G_w.T, G_b[None, :], gs)
    return out[:N]


# speedup vs baseline: 6.7297x; 1.2965x over previous
import functools

import jax
import jax.numpy as jnp
from jax import lax
from jax.experimental import pallas as pl
from jax.experimental.pallas import tpu as pltpu
from jax.experimental.pallas import tpu_sc as plsc

N = 10000
E = 320000
D = 128
N_PAD = 10240
BLK = 512
NC = 2
NS = 16
NW = NC * NS
CHUNK = 40
NBUF = 3
L = 16
E_PER_W = E // NW
N_IT = E_PER_W // CHUNK
DCH = 2000

_SC_PARAMS = pltpu.CompilerParams(needs_layout_passes=False)
_MESH = plsc.VectorSubcoreMesh(core_axis_name="c", subcore_axis_name="s")


@functools.partial(
    pl.kernel, mesh=_MESH,
    out_type=jax.ShapeDtypeStruct((NW, N_PAD), jnp.float32),
    scratch_types=[
        pltpu.VMEM((1, DCH), jnp.int32),
        pltpu.VMEM((1, DCH), jnp.int32),
        pltpu.VMEM((N_PAD,), jnp.float32),
        pltpu.SemaphoreType.DMA,
        pltpu.SemaphoreType.DMA,
    ],
    compiler_params=_SC_PARAMS)
def _deg_hist(dst_hbm, z1_hbm, deg_hbm, d0, d1, hist_v, sem0, sem1):
    wid = lax.axis_index("c") * NS + lax.axis_index("s")
    n_dch = E_PER_W // DCH
    dbufs = [d0, d1]
    sems = [sem0, sem1]
    pltpu.async_copy(dst_hbm.at[wid, 0], d0, sem0)
    pltpu.sync_copy(z1_hbm, hist_v)
    ones16 = jnp.full((L,), 1.0, jnp.float32)

    for c in range(n_dch):
        b = c % 2
        pltpu.make_async_copy(dst_hbm.at[wid, c], dbufs[b], sems[b]).wait()
        if c + 1 < n_dch:
            pltpu.async_copy(dst_hbm.at[wid, c + 1], dbufs[1 - b],
                             sems[1 - b])

        def body(i, carry, b=b):
            idx16 = dbufs[b][0, pl.ds(i * L, L)]
            plsc.addupdate_scatter(hist_v, [idx16], ones16)
            return carry

        lax.fori_loop(0, DCH // L, body, 0)
    pltpu.sync_copy(hist_v, deg_hbm.at[wid])


@functools.partial(
    pl.kernel, mesh=_MESH,
    out_type=jax.ShapeDtypeStruct((NC * N_PAD, D), jnp.float32),
    scratch_types=(
        [pltpu.VMEM((2, CHUNK), jnp.int32) for _ in range(NBUF)]
        + [pltpu.VMEM((CHUNK, D), jnp.float32) for _ in range(NBUF)]
        + [pltpu.SemaphoreType.DMA for _ in range(2 * NBUF)]
        + [pltpu.SemaphoreType.DMA,
           pltpu.VMEM_SHARED((N_PAD, D), jnp.float32)]
    ),
    compiler_params=_SC_PARAMS)
def _edge_scatter(sd_hbm, m_hbm, z2_hbm, out_hbm, *rest):
    idxb = rest[:NBUF]
    bufs = rest[NBUF:2 * NBUF]
    isems = rest[2 * NBUF:3 * NBUF]
    gsems = rest[3 * NBUF:4 * NBUF]
    zsem, agg_s = rest[4 * NBUF:]
    cid = lax.axis_index("c")
    sid = lax.axis_index("s")
    wid = cid * NS + sid
    rows_ps = N_PAD // NS
    r0 = sid * rows_ps

    def idx_load(chunk, b):
        pltpu.async_copy(sd_hbm.at[wid, chunk], idxb[b], isems[b])

    def idx_wait(chunk, b):
        pltpu.make_async_copy(sd_hbm.at[wid, chunk], idxb[b], isems[b]).wait()

    def gather(b):
        pltpu.async_copy(m_hbm.at[idxb[b].at[0]], bufs[b], gsems[b])

    def gather_wait(b):
        pltpu.make_async_copy(m_hbm.at[idxb[b].at[0]], bufs[b],
                              gsems[b]).wait()

    zcp = pltpu.async_copy(z2_hbm.at[pl.ds(r0, rows_ps)],
                           agg_s.at[pl.ds(r0, rows_ps)], zsem)
    for b in range(NBUF):
        idx_load(b, b)
    for b in range(NBUF - 1):
        idx_wait(b, b)
        gather(b)
    zcp.wait()
    plsc.subcore_barrier()

    def visit(k, b):
        b2 = (b + NBUF - 1) % NBUF

        @pl.when(k + NBUF - 1 < N_IT)
        def _():
            idx_wait(k + NBUF - 1, b2)
            gather(b2)

        @pl.when(k < N_IT)
        def _():
            gather_wait(b)
            pltpu.sync_copy(bufs[b], agg_s.at[idxb[b].at[1]], add=True)

        @pl.when(k + NBUF < N_IT)
        def _():
            idx_load(k + NBUF, b)

    def round_body(r, carry):
        for b in range(NBUF):
            visit(r * NBUF + b, b)
        return carry

    n_rounds = (N_IT + NBUF - 1) // NBUF
    lax.fori_loop(0, n_rounds, round_body, 0)
    plsc.subcore_barrier()

    pltpu.sync_copy(agg_s.at[pl.ds(r0, rows_ps)],
                    out_hbm.at[pl.ds(cid * N_PAD + r0, rows_ps)])


def _pre1_body(h_ref, qt_ref, qb_ref, o_ref):
    m = jnp.dot(h_ref[...], qt_ref[...], preferred_element_type=jnp.float32)
    o_ref[...] = jnp.maximum(m + qb_ref[...], 0.0)


def _mid_body(h_ref, a0_ref, a1_ref, degp_ref, w1a_ref, w1b_ref, w1b_b_ref,
              q2t_ref, q2b_ref, h1_ref, m2_ref, deg_ref):
    deg = jnp.sum(degp_ref[...], axis=0)
    hn = (a0_ref[...] + a1_ref[...]) / jnp.maximum(deg, 1.0)[:, None]
    z = jnp.dot(h_ref[...], w1a_ref[...], preferred_element_type=jnp.float32)
    z = z + jnp.dot(hn, w1b_ref[...], preferred_element_type=jnp.float32)
    z = jnp.maximum(z + w1b_b_ref[...], 0.0)
    nrm = jnp.sqrt(jnp.sum(z * z, axis=1, keepdims=True))
    h1 = z / (nrm + 1e-6)
    h1_ref[...] = h1
    m2 = jnp.dot(h1, q2t_ref[...], preferred_element_type=jnp.float32)
    m2_ref[...] = jnp.maximum(m2 + q2b_ref[...], 0.0)
    deg_ref[...] = deg


def _post_body(h1_ref, a0_ref, a1_ref, deg_ref, w2a_ref, w2b_ref, w2b_b_ref,
               gt_ref, gb_ref, gs_ref, o_ref):
    deg = deg_ref[...]
    hn = (a0_ref[...] + a1_ref[...]) / jnp.maximum(deg, 1.0)[:, None]
    z = jnp.dot(h1_ref[...], w2a_ref[...], preferred_element_type=jnp.float32)
    z = z + jnp.dot(hn, w2b_ref[...], preferred_element_type=jnp.float32)
    z = jnp.maximum(z + w2b_b_ref[...], 0.0)
    nrm = jnp.sqrt(jnp.sum(z * z, axis=1, keepdims=True))
    h2 = z / (nrm + 1e-6)
    out = jnp.dot(h2, gt_ref[...], preferred_element_type=jnp.float32)
    o_ref[...] = gs_ref[...] * jnp.maximum(out + gb_ref[...], 0.0)


def _row_spec(width):
    return pl.BlockSpec((BLK, width), lambda i: (i, 0))


def _full_spec(shape):
    return pl.BlockSpec(shape, lambda i: tuple(0 for _ in shape))


_GRID = N_PAD // BLK
_vec_spec = pl.BlockSpec((BLK,), lambda i: (i,))

_pre1 = pl.pallas_call(
    _pre1_body,
    grid=(_GRID,),
    in_specs=[_row_spec(D), _full_spec((D, D)), _full_spec((1, D))],
    out_specs=_row_spec(D),
    out_shape=jax.ShapeDtypeStruct((N_PAD, D), jnp.float32),
)

_mid = pl.pallas_call(
    _mid_body,
    grid=(_GRID,),
    in_specs=[_row_spec(D), _row_spec(D), _row_spec(D),
              pl.BlockSpec((NW, BLK), lambda i: (0, i)),
              _full_spec((D, D)), _full_spec((D, D)), _full_spec((1, D)),
              _full_spec((D, D)), _full_spec((1, D))],
    out_specs=[_row_spec(D), _row_spec(D), _vec_spec],
    out_shape=[jax.ShapeDtypeStruct((N_PAD, D), jnp.float32),
               jax.ShapeDtypeStruct((N_PAD, D), jnp.float32),
               jax.ShapeDtypeStruct((N_PAD,), jnp.float32)],
)

_post = pl.pallas_call(
    _post_body,
    grid=(_GRID,),
    in_specs=[_row_spec(D), _row_spec(D), _row_spec(D), _vec_spec,
              _full_spec((D, D)), _full_spec((D, D)), _full_spec((1, D)),
              _full_spec((D, D)), _full_spec((1, D)), _full_spec((1, D))],
    out_specs=_row_spec(D),
    out_shape=jax.ShapeDtypeStruct((N_PAD, D), jnp.float32),
)


@jax.jit
def kernel(g, h, Q1_w, Q1_b, W1_w, W1_b, Q2_w, Q2_b, W2_w, W2_b, G_w, G_b,
           g_scale):
    gi = g.astype(jnp.int32)
    sd4d = jnp.stack([gi[0].reshape(NW, N_IT, CHUNK),
                      gi[1].reshape(NW, N_IT, CHUNK)], axis=2)
    dst4d = gi[1].reshape(NW, E_PER_W // DCH, 1, DCH)
    h_pad = jnp.zeros((N_PAD, D), jnp.float32).at[:N].set(h)
    zeros_nd = jnp.zeros((N_PAD, D), jnp.float32)
    zeros_1d = jnp.zeros((N_PAD,), jnp.float32)

    degp = _deg_hist(dst4d, zeros_1d)

    m1 = _pre1(h_pad, Q1_w.T, Q1_b[None, :])
    agg1 = _edge_scatter(sd4d, m1, zeros_nd)

    h1, m2, deg = _mid(h_pad, agg1[:N_PAD], agg1[N_PAD:], degp,
                       W1_w[:, :D].T, W1_w[:, D:].T, W1_b[None, :],
                       Q2_w.T, Q2_b[None, :])

    agg2 = _edge_scatter(sd4d, m2, zeros_nd)

    gs = jnp.broadcast_to(g_scale.astype(jnp.float32), (1, D))
    out = _post(h1, agg2[:N_PAD], agg2[N_PAD:], deg,
                W2_w[:, :D].T, W2_w[:, D:].T, W2_b[None, :],
                G_w.T, G_b[None, :], gs)
    return out[:N]


# speedup vs baseline: 8.4428x; 1.2546x over previous
import functools

import jax
import jax.numpy as jnp
from jax import lax
from jax.experimental import pallas as pl
from jax.experimental.pallas import tpu as pltpu
from jax.experimental.pallas import tpu_sc as plsc

N = 10000
E = 320000
D = 128
N_PAD = 10240
BLK = 512
NC = 2
NS = 16
NW = NC * NS
CHUNK = 40
NBUF = 3
L = 16
E_PER_W = E // NW
N_IT = E_PER_W // CHUNK
DCH = 2000

_SC_PARAMS = pltpu.CompilerParams(needs_layout_passes=False)
_MESH = plsc.VectorSubcoreMesh(core_axis_name="c", subcore_axis_name="s")


@functools.partial(
    pl.kernel, mesh=_MESH,
    out_type=jax.ShapeDtypeStruct((NW, N_PAD), jnp.float32),
    scratch_types=[
        pltpu.VMEM((1, DCH), jnp.int32),
        pltpu.VMEM((1, DCH), jnp.int32),
        pltpu.VMEM((N_PAD,), jnp.float32),
        pltpu.SemaphoreType.DMA,
        pltpu.SemaphoreType.DMA,
    ],
    compiler_params=_SC_PARAMS)
def _deg_hist(dst_hbm, z1_hbm, deg_hbm, d0, d1, hist_v, sem0, sem1):
    wid = lax.axis_index("c") * NS + lax.axis_index("s")
    n_dch = E_PER_W // DCH
    dbufs = [d0, d1]
    sems = [sem0, sem1]
    pltpu.async_copy(dst_hbm.at[wid, 0], d0, sem0)
    pltpu.sync_copy(z1_hbm, hist_v)
    ones16 = jnp.full((L,), 1.0, jnp.float32)

    for c in range(n_dch):
        b = c % 2
        pltpu.make_async_copy(dst_hbm.at[wid, c], dbufs[b], sems[b]).wait()
        if c + 1 < n_dch:
            pltpu.async_copy(dst_hbm.at[wid, c + 1], dbufs[1 - b],
                             sems[1 - b])

        def body(i, carry, b=b):
            idx16 = dbufs[b][0, pl.ds(i * L, L)]
            plsc.addupdate_scatter(hist_v, [idx16], ones16)
            return carry

        lax.fori_loop(0, DCH // L, body, 0)
    pltpu.sync_copy(hist_v, deg_hbm.at[wid])


@functools.partial(
    pl.kernel, mesh=_MESH,
    out_type=jax.ShapeDtypeStruct((NC * N_PAD, D), jnp.float32),
    scratch_types=(
        [pltpu.VMEM((2, CHUNK), jnp.int32) for _ in range(2 * NBUF)]
        + [pltpu.VMEM((CHUNK, D), jnp.float32) for _ in range(NBUF)]
        + [pltpu.SemaphoreType.DMA for _ in range(2 * NBUF)]
        + [pltpu.SemaphoreType.DMA for _ in range(2 * NBUF)]
        + [pltpu.SemaphoreType.DMA,
           pltpu.VMEM_SHARED((N_PAD, D), jnp.float32)]
    ),
    compiler_params=_SC_PARAMS)
def _edge_scatter(sd_hbm, m_hbm, z2_hbm, out_hbm, *rest):
    NI = 2 * NBUF
    idxb = rest[:NI]
    bufs = rest[NI:NI + NBUF]
    isems = rest[NI + NBUF:2 * NI + NBUF]
    gsems = rest[2 * NI + NBUF:2 * NI + 2 * NBUF]
    ssems = rest[2 * NI + 2 * NBUF:2 * NI + 3 * NBUF]
    zsem, agg_s = rest[2 * NI + 3 * NBUF:]
    cid = lax.axis_index("c")
    sid = lax.axis_index("s")
    wid = cid * NS + sid
    rows_ps = N_PAD // NS
    r0 = sid * rows_ps

    def idx_load(chunk, i):
        pltpu.async_copy(sd_hbm.at[wid, chunk], idxb[i], isems[i])

    def idx_wait(chunk, i):
        pltpu.make_async_copy(sd_hbm.at[wid, chunk], idxb[i], isems[i]).wait()

    def gather(b, i):
        pltpu.async_copy(m_hbm.at[idxb[i].at[0]], bufs[b], gsems[b])

    def gather_wait(b, i):
        pltpu.make_async_copy(m_hbm.at[idxb[i].at[0]], bufs[b],
                              gsems[b]).wait()

    def scatter(b, i):
        pltpu.async_copy(bufs[b], agg_s.at[idxb[i].at[1]], ssems[b],
                         add=True)

    def scatter_wait(b, i):
        pltpu.make_async_copy(bufs[b], agg_s.at[idxb[i].at[1]],
                              ssems[b]).wait()

    zcp = pltpu.async_copy(z2_hbm.at[pl.ds(r0, rows_ps)],
                           agg_s.at[pl.ds(r0, rows_ps)], zsem)
    for b in range(NBUF):
        idx_load(b, b)
    for b in range(NBUF - 1):
        idx_wait(b, b)
        gather(b, b)
    zcp.wait()
    plsc.subcore_barrier()

    def visit(k, b, ib):
        b2 = (b + NBUF - 1) % NBUF
        ib_g = (ib + NBUF - 1) % NI
        ib_l = (ib + NBUF) % NI

        @pl.when(k < N_IT)
        def _():
            gather_wait(b, ib)
            scatter(b, ib)

        @pl.when((k >= 1) & (k <= N_IT))
        def _():
            scatter_wait(b2, (ib + NI - 1) % NI)

        @pl.when(k + NBUF - 1 < N_IT)
        def _():
            idx_wait(k + NBUF - 1, ib_g)
            gather(b2, ib_g)

        @pl.when(k + NBUF < N_IT)
        def _():
            idx_load(k + NBUF, ib_l)

    def round_body(r, carry):
        for j in range(NI):
            k = r * NI + j
            visit(k, j % NBUF, j)
        return carry

    n_rounds = (N_IT + NI) // NI
    lax.fori_loop(0, n_rounds, round_body, 0)
    plsc.subcore_barrier()

    pltpu.sync_copy(agg_s.at[pl.ds(r0, rows_ps)],
                    out_hbm.at[pl.ds(cid * N_PAD + r0, rows_ps)])


def _pre1_body(h_ref, qt_ref, qb_ref, o_ref):
    m = jnp.dot(h_ref[...], qt_ref[...], preferred_element_type=jnp.float32)
    o_ref[...] = jnp.maximum(m + qb_ref[...], 0.0)


def _mid_body(h_ref, a0_ref, a1_ref, degp_ref, w1a_ref, w1b_ref, w1b_b_ref,
              q2t_ref, q2b_ref, h1_ref, m2_ref, deg_ref):
    deg = jnp.sum(degp_ref[...], axis=0)
    hn = (a0_ref[...] + a1_ref[...]) / jnp.maximum(deg, 1.0)[:, None]
    z = jnp.dot(h_ref[...], w1a_ref[...], preferred_element_type=jnp.float32)
    z = z + jnp.dot(hn, w1b_ref[...], preferred_element_type=jnp.float32)
    z = jnp.maximum(z + w1b_b_ref[...], 0.0)
    nrm = jnp.sqrt(jnp.sum(z * z, axis=1, keepdims=True))
    h1 = z / (nrm + 1e-6)
    h1_ref[...] = h1
    m2 = jnp.dot(h1, q2t_ref[...], preferred_element_type=jnp.float32)
    m2_ref[...] = jnp.maximum(m2 + q2b_ref[...], 0.0)
    deg_ref[...] = deg


def _post_body(h1_ref, a0_ref, a1_ref, deg_ref, w2a_ref, w2b_ref, w2b_b_ref,
               gt_ref, gb_ref, gs_ref, o_ref):
    deg = deg_ref[...]
    hn = (a0_ref[...] + a1_ref[...]) / jnp.maximum(deg, 1.0)[:, None]
    z = jnp.dot(h1_ref[...], w2a_ref[...], preferred_element_type=jnp.float32)
    z = z + jnp.dot(hn, w2b_ref[...], preferred_element_type=jnp.float32)
    z = jnp.maximum(z + w2b_b_ref[...], 0.0)
    nrm = jnp.sqrt(jnp.sum(z * z, axis=1, keepdims=True))
    h2 = z / (nrm + 1e-6)
    out = jnp.dot(h2, gt_ref[...], preferred_element_type=jnp.float32)
    o_ref[...] = gs_ref[...] * jnp.maximum(out + gb_ref[...], 0.0)


def _row_spec(width):
    return pl.BlockSpec((BLK, width), lambda i: (i, 0))


def _full_spec(shape):
    return pl.BlockSpec(shape, lambda i: tuple(0 for _ in shape))


_GRID = N_PAD // BLK
_vec_spec = pl.BlockSpec((BLK,), lambda i: (i,))

_pre1 = pl.pallas_call(
    _pre1_body,
    grid=(_GRID,),
    in_specs=[_row_spec(D), _full_spec((D, D)), _full_spec((1, D))],
    out_specs=_row_spec(D),
    out_shape=jax.ShapeDtypeStruct((N_PAD, D), jnp.float32),
)

_mid = pl.pallas_call(
    _mid_body,
    grid=(_GRID,),
    in_specs=[_row_spec(D), _row_spec(D), _row_spec(D),
              pl.BlockSpec((NW, BLK), lambda i: (0, i)),
              _full_spec((D, D)), _full_spec((D, D)), _full_spec((1, D)),
              _full_spec((D, D)), _full_spec((1, D))],
    out_specs=[_row_spec(D), _row_spec(D), _vec_spec],
    out_shape=[jax.ShapeDtypeStruct((N_PAD, D), jnp.float32),
               jax.ShapeDtypeStruct((N_PAD, D), jnp.float32),
               jax.ShapeDtypeStruct((N_PAD,), jnp.float32)],
)

_post = pl.pallas_call(
    _post_body,
    grid=(_GRID,),
    in_specs=[_row_spec(D), _row_spec(D), _row_spec(D), _vec_spec,
              _full_spec((D, D)), _full_spec((D, D)), _full_spec((1, D)),
              _full_spec((D, D)), _full_spec((1, D)), _full_spec((1, D))],
    out_specs=_row_spec(D),
    out_shape=jax.ShapeDtypeStruct((N_PAD, D), jnp.float32),
)


@jax.jit
def kernel(g, h, Q1_w, Q1_b, W1_w, W1_b, Q2_w, Q2_b, W2_w, W2_b, G_w, G_b,
           g_scale):
    gi = g.astype(jnp.int32)
    sd4d = jnp.stack([gi[0].reshape(NW, N_IT, CHUNK),
                      gi[1].reshape(NW, N_IT, CHUNK)], axis=2)
    dst4d = gi[1].reshape(NW, E_PER_W // DCH, 1, DCH)
    h_pad = jnp.zeros((N_PAD, D), jnp.float32).at[:N].set(h)
    zeros_nd = jnp.zeros((N_PAD, D), jnp.float32)
    zeros_1d = jnp.zeros((N_PAD,), jnp.float32)

    degp = _deg_hist(dst4d, zeros_1d)

    m1 = _pre1(h_pad, Q1_w.T, Q1_b[None, :])
    agg1 = _edge_scatter(sd4d, m1, zeros_nd)

    h1, m2, deg = _mid(h_pad, agg1[:N_PAD], agg1[N_PAD:], degp,
                       W1_w[:, :D].T, W1_w[:, D:].T, W1_b[None, :],
                       Q2_w.T, Q2_b[None, :])

    agg2 = _edge_scatter(sd4d, m2, zeros_nd)

    gs = jnp.broadcast_to(g_scale.astype(jnp.float32), (1, D))
    out = _post(h1, agg2[:N_PAD], agg2[N_PAD:], deg,
                W2_w[:, :D].T, W2_w[:, D:].T, W2_b[None, :],
                G_w.T, G_b[None, :], gs)
    return out[:N]


# speedup vs baseline: 8.5806x; 1.0163x over previous
import functools

import jax
import jax.numpy as jnp
from jax import lax
from jax.experimental import pallas as pl
from jax.experimental.pallas import tpu as pltpu
from jax.experimental.pallas import tpu_sc as plsc

N = 10000
E = 320000
D = 128
N_PAD = 10240
BLK = 512
NC = 2
NS = 16
NW = NC * NS
CHUNK = 40
NBUF = 3
L = 16
E_PER_W = E // NW
N_IT = E_PER_W // CHUNK

_SC_PARAMS = pltpu.CompilerParams(needs_layout_passes=False)
_MESH = plsc.VectorSubcoreMesh(core_axis_name="c", subcore_axis_name="s")


def _make_edge_scatter(with_deg):
  out_type = [jax.ShapeDtypeStruct((NC * N_PAD, D), jnp.float32)]
  scratch = (
      [pltpu.VMEM((2, CHUNK), jnp.int32) for _ in range(2 * NBUF)]
      + [pltpu.VMEM((CHUNK, D), jnp.float32) for _ in range(NBUF)]
      + [pltpu.SemaphoreType.DMA for _ in range(2 * NBUF)]
      + [pltpu.SemaphoreType.DMA for _ in range(2 * NBUF)]
      + [pltpu.SemaphoreType.DMA]
  )
  if with_deg:
      out_type.append(jax.ShapeDtypeStruct((NW, N_PAD), jnp.float32))
      scratch.append(pltpu.VMEM((N_PAD,), jnp.float32))
  scratch.append(pltpu.VMEM_SHARED((N_PAD, D), jnp.float32))

  @functools.partial(pl.kernel, mesh=_MESH, out_type=out_type,
                     scratch_types=scratch, compiler_params=_SC_PARAMS)
  def _edge_scatter(*args):
    NI = 2 * NBUF
    if with_deg:
        sd_hbm, m_hbm, z2_hbm, z1_hbm, out_hbm, deg_hbm, *rest = args
    else:
        sd_hbm, m_hbm, z2_hbm, out_hbm, *rest = args
    idxb = rest[:NI]
    bufs = rest[NI:NI + NBUF]
    isems = rest[NI + NBUF:2 * NI + NBUF]
    gsems = rest[2 * NI + NBUF:2 * NI + 2 * NBUF]
    ssems = rest[2 * NI + 2 * NBUF:2 * NI + 3 * NBUF]
    if with_deg:
        zsem, hist_v, agg_s = rest[2 * NI + 3 * NBUF:]
    else:
        zsem, agg_s = rest[2 * NI + 3 * NBUF:]
    cid = lax.axis_index("c")
    sid = lax.axis_index("s")
    wid = cid * NS + sid
    rows_ps = N_PAD // NS
    r0 = sid * rows_ps

    def idx_load(chunk, i):
        pltpu.async_copy(sd_hbm.at[wid, chunk], idxb[i], isems[i])

    def idx_wait(chunk, i):
        pltpu.make_async_copy(sd_hbm.at[wid, chunk], idxb[i], isems[i]).wait()

    def gather(b, i):
        pltpu.async_copy(m_hbm.at[idxb[i].at[0]], bufs[b], gsems[b])

    def gather_wait(b, i):
        pltpu.make_async_copy(m_hbm.at[idxb[i].at[0]], bufs[b],
                              gsems[b]).wait()

    def scatter(b, i):
        pltpu.async_copy(bufs[b], agg_s.at[idxb[i].at[1]], ssems[b],
                         add=True)

    def scatter_wait(b, i):
        pltpu.make_async_copy(bufs[b], agg_s.at[idxb[i].at[1]],
                              ssems[b]).wait()

    zcp = pltpu.async_copy(z2_hbm.at[pl.ds(r0, rows_ps)],
                           agg_s.at[pl.ds(r0, rows_ps)], zsem)
    for b in range(NBUF):
        idx_load(b, b)
    for b in range(NBUF - 1):
        idx_wait(b, b)
        gather(b, b)
    if with_deg:
        pltpu.sync_copy(z1_hbm, hist_v)
    zcp.wait()
    plsc.subcore_barrier()

    ones16 = jnp.full((L,), 1.0, jnp.float32)
    tailmask = lax.iota(jnp.int32, L) >= (L - CHUNK % L) if CHUNK % L else None

    def hist_chunk(ib):
        for j in range(CHUNK // L):
            plsc.addupdate_scatter(hist_v, [idxb[ib][1, pl.ds(j * L, L)]],
                                   ones16)
        if CHUNK % L:
            plsc.addupdate_scatter(hist_v,
                                   [idxb[ib][1, pl.ds(CHUNK - L, L)]],
                                   ones16, mask=tailmask)

    def visit(k, b, ib):
        b2 = (b + NBUF - 1) % NBUF
        ib_g = (ib + NBUF - 1) % NI
        ib_l = (ib + NBUF) % NI

        @pl.when(k < N_IT)
        def _():
            gather_wait(b, ib)
            scatter(b, ib)
            if with_deg:
                hist_chunk(ib)

        @pl.when((k >= 1) & (k <= N_IT))
        def _():
            scatter_wait(b2, (ib + NI - 1) % NI)

        @pl.when(k + NBUF - 1 < N_IT)
        def _():
            idx_wait(k + NBUF - 1, ib_g)
            gather(b2, ib_g)

        @pl.when(k + NBUF < N_IT)
        def _():
            idx_load(k + NBUF, ib_l)

    def round_body(r, carry):
        for j in range(NI):
            k = r * NI + j
            visit(k, j % NBUF, j)
        return carry

    n_rounds = (N_IT + NI) // NI
    lax.fori_loop(0, n_rounds, round_body, 0)
    if with_deg:
        pltpu.sync_copy(hist_v, deg_hbm.at[wid])
    plsc.subcore_barrier()

    pltpu.sync_copy(agg_s.at[pl.ds(r0, rows_ps)],
                    out_hbm.at[pl.ds(cid * N_PAD + r0, rows_ps)])

  return _edge_scatter


_edge_scatter_l1 = _make_edge_scatter(True)
_edge_scatter_l2 = _make_edge_scatter(False)


def _pre1_body(h_ref, qt_ref, qb_ref, o_ref):
    m = jnp.dot(h_ref[...], qt_ref[...], preferred_element_type=jnp.float32)
    o_ref[...] = jnp.maximum(m + qb_ref[...], 0.0)


def _mid_body(h_ref, a0_ref, a1_ref, degp_ref, w1a_ref, w1b_ref, w1b_b_ref,
              q2t_ref, q2b_ref, h1_ref, m2_ref, deg_ref):
    deg = jnp.sum(degp_ref[...], axis=0)
    hn = (a0_ref[...] + a1_ref[...]) / jnp.maximum(deg, 1.0)[:, None]
    z = jnp.dot(h_ref[...], w1a_ref[...], preferred_element_type=jnp.float32)
    z = z + jnp.dot(hn, w1b_ref[...], preferred_element_type=jnp.float32)
    z = jnp.maximum(z + w1b_b_ref[...], 0.0)
    nrm = jnp.sqrt(jnp.sum(z * z, axis=1, keepdims=True))
    h1 = z / (nrm + 1e-6)
    h1_ref[...] = h1
    m2 = jnp.dot(h1, q2t_ref[...], preferred_element_type=jnp.float32)
    m2_ref[...] = jnp.maximum(m2 + q2b_ref[...], 0.0)
    deg_ref[...] = deg


def _post_body(h1_ref, a0_ref, a1_ref, deg_ref, w2a_ref, w2b_ref, w2b_b_ref,
               gt_ref, gb_ref, gs_ref, o_ref):
    deg = deg_ref[...]
    hn = (a0_ref[...] + a1_ref[...]) / jnp.maximum(deg, 1.0)[:, None]
    z = jnp.dot(h1_ref[...], w2a_ref[...], preferred_element_type=jnp.float32)
    z = z + jnp.dot(hn, w2b_ref[...], preferred_element_type=jnp.float32)
    z = jnp.maximum(z + w2b_b_ref[...], 0.0)
    nrm = jnp.sqrt(jnp.sum(z * z, axis=1, keepdims=True))
    h2 = z / (nrm + 1e-6)
    out = jnp.dot(h2, gt_ref[...], preferred_element_type=jnp.float32)
    o_ref[...] = gs_ref[...] * jnp.maximum(out + gb_ref[...], 0.0)


def _row_spec(width):
    return pl.BlockSpec((BLK, width), lambda i: (i, 0))


def _full_spec(shape):
    return pl.BlockSpec(shape, lambda i: tuple(0 for _ in shape))


_GRID = N_PAD // BLK
_vec_spec = pl.BlockSpec((BLK,), lambda i: (i,))

_pre1 = pl.pallas_call(
    _pre1_body,
    grid=(_GRID,),
    in_specs=[_row_spec(D), _full_spec((D, D)), _full_spec((1, D))],
    out_specs=_row_spec(D),
    out_shape=jax.ShapeDtypeStruct((N_PAD, D), jnp.float32),
)

_mid = pl.pallas_call(
    _mid_body,
    grid=(_GRID,),
    in_specs=[_row_spec(D), _row_spec(D), _row_spec(D),
              pl.BlockSpec((NW, BLK), lambda i: (0, i)),
              _full_spec((D, D)), _full_spec((D, D)), _full_spec((1, D)),
              _full_spec((D, D)), _full_spec((1, D))],
    out_specs=[_row_spec(D), _row_spec(D), _vec_spec],
    out_shape=[jax.ShapeDtypeStruct((N_PAD, D), jnp.float32),
               jax.ShapeDtypeStruct((N_PAD, D), jnp.float32),
               jax.ShapeDtypeStruct((N_PAD,), jnp.float32)],
)

_post = pl.pallas_call(
    _post_body,
    grid=(_GRID,),
    in_specs=[_row_spec(D), _row_spec(D), _row_spec(D), _vec_spec,
              _full_spec((D, D)), _full_spec((D, D)), _full_spec((1, D)),
              _full_spec((D, D)), _full_spec((1, D)), _full_spec((1, D))],
    out_specs=_row_spec(D),
    out_shape=jax.ShapeDtypeStruct((N_PAD, D), jnp.float32),
)


@jax.jit
def kernel(g, h, Q1_w, Q1_b, W1_w, W1_b, Q2_w, Q2_b, W2_w, W2_b, G_w, G_b,
           g_scale):
    gi = g.astype(jnp.int32)
    sd4d = jnp.stack([gi[0].reshape(NW, N_IT, CHUNK),
                      gi[1].reshape(NW, N_IT, CHUNK)], axis=2)

    h_pad = jnp.zeros((N_PAD, D), jnp.float32).at[:N].set(h)
    zeros_nd = jnp.zeros((N_PAD, D), jnp.float32)
    zeros_1d = jnp.zeros((N_PAD,), jnp.float32)

    m1 = _pre1(h_pad, Q1_w.T, Q1_b[None, :])
    agg1, degp = _edge_scatter_l1(sd4d, m1, zeros_nd, zeros_1d)

    h1, m2, deg = _mid(h_pad, agg1[:N_PAD], agg1[N_PAD:], degp,
                       W1_w[:, :D].T, W1_w[:, D:].T, W1_b[None, :],
                       Q2_w.T, Q2_b[None, :])

    agg2, = _edge_scatter_l2(sd4d, m2, zeros_nd)

    gs = jnp.broadcast_to(g_scale.astype(jnp.float32), (1, D))
    out = _post(h1, agg2[:N_PAD], agg2[N_PAD:], deg,
                W2_w[:, :D].T, W2_w[:, D:].T, W2_b[None, :],
                G_w.T, G_b[None, :], gs)
    return out[:N]


# speedup vs baseline: 9.0247x; 1.0518x over previous
import functools

import jax
import jax.numpy as jnp
from jax import lax
from jax.experimental import pallas as pl
from jax.experimental.pallas import tpu as pltpu
from jax.experimental.pallas import tpu_sc as plsc

N = 10000
E = 320000
D = 128
N_PAD = 10240
BLK = 512
NC = 2
NS = 16
NW = NC * NS
CHUNK = 40
NBUF = 3
L = 16
E_PER_W = E // NW
N_IT = E_PER_W // CHUNK

_SC_PARAMS = pltpu.CompilerParams(needs_layout_passes=False)
_MESH = plsc.VectorSubcoreMesh(core_axis_name="c", subcore_axis_name="s")


def _make_edge_scatter(with_deg):
  out_type = [jax.ShapeDtypeStruct((NC * N_PAD, D), jnp.float32)]
  scratch = (
      [pltpu.VMEM((2, CHUNK), jnp.int32) for _ in range(2 * NBUF)]
      + [pltpu.VMEM((CHUNK, D), jnp.float32) for _ in range(NBUF)]
      + [pltpu.SemaphoreType.DMA for _ in range(2 * NBUF)]
      + [pltpu.SemaphoreType.DMA for _ in range(2 * NBUF)]
      + [pltpu.SemaphoreType.DMA]
  )
  if with_deg:
      out_type.append(jax.ShapeDtypeStruct((NW, N_PAD), jnp.float32))
      scratch.append(pltpu.VMEM((N_PAD,), jnp.float32))
  scratch.append(pltpu.VMEM_SHARED((N_PAD, D), jnp.float32))

  @functools.partial(pl.kernel, mesh=_MESH, out_type=out_type,
                     scratch_types=scratch, compiler_params=_SC_PARAMS)
  def _edge_scatter(*args):
    NI = 2 * NBUF
    if with_deg:
        sd_hbm, m_hbm, z2_hbm, z1_hbm, out_hbm, deg_hbm, *rest = args
    else:
        sd_hbm, m_hbm, z2_hbm, out_hbm, *rest = args
    idxb = rest[:NI]
    bufs = rest[NI:NI + NBUF]
    isems = rest[NI + NBUF:2 * NI + NBUF]
    gsems = rest[2 * NI + NBUF:2 * NI + 2 * NBUF]
    ssems = rest[2 * NI + 2 * NBUF:2 * NI + 3 * NBUF]
    if with_deg:
        zsem, hist_v, agg_s = rest[2 * NI + 3 * NBUF:]
    else:
        zsem, agg_s = rest[2 * NI + 3 * NBUF:]
    cid = lax.axis_index("c")
    sid = lax.axis_index("s")
    wid = cid * NS + sid
    rows_ps = N_PAD // NS
    r0 = sid * rows_ps

    def idx_load(chunk, i):
        pltpu.async_copy(sd_hbm.at[wid, chunk], idxb[i], isems[i])

    def idx_wait(chunk, i):
        pltpu.make_async_copy(sd_hbm.at[wid, chunk], idxb[i], isems[i]).wait()

    def gather(b, i):
        pltpu.async_copy(m_hbm.at[idxb[i].at[0]], bufs[b], gsems[b])

    def gather_wait(b, i):
        pltpu.make_async_copy(m_hbm.at[idxb[i].at[0]], bufs[b],
                              gsems[b]).wait()

    def scatter(b, i):
        pltpu.async_copy(bufs[b], agg_s.at[idxb[i].at[1]], ssems[b],
                         add=True)

    def scatter_wait(b, i):
        pltpu.make_async_copy(bufs[b], agg_s.at[idxb[i].at[1]],
                              ssems[b]).wait()

    zcp = pltpu.async_copy(z2_hbm.at[pl.ds(r0, rows_ps)],
                           agg_s.at[pl.ds(r0, rows_ps)], zsem)
    for b in range(NBUF):
        idx_load(b, b)
    for b in range(NBUF - 1):
        idx_wait(b, b)
        gather(b, b)
    if with_deg:
        pltpu.sync_copy(z1_hbm, hist_v)
    zcp.wait()
    plsc.subcore_barrier()

    ones16 = jnp.full((L,), 1.0, jnp.float32)
    tailmask = lax.iota(jnp.int32, L) >= (L - CHUNK % L) if CHUNK % L else None

    def hist_chunk(ib):
        for j in range(CHUNK // L):
            plsc.addupdate_scatter(hist_v, [idxb[ib][1, pl.ds(j * L, L)]],
                                   ones16)
        if CHUNK % L:
            plsc.addupdate_scatter(hist_v,
                                   [idxb[ib][1, pl.ds(CHUNK - L, L)]],
                                   ones16, mask=tailmask)

    def visit(k, b, ib):
        b2 = (b + NBUF - 1) % NBUF
        ib_g = (ib + NBUF - 1) % NI
        ib_l = (ib + NBUF) % NI

        @pl.when(k < N_IT)
        def _():
            gather_wait(b, ib)
            scatter(b, ib)
            if with_deg:
                hist_chunk(ib)

        @pl.when((k >= 1) & (k <= N_IT))
        def _():
            scatter_wait(b2, (ib + NI - 1) % NI)

        @pl.when(k + NBUF - 1 < N_IT)
        def _():
            idx_wait(k + NBUF - 1, ib_g)
            gather(b2, ib_g)

        @pl.when(k + NBUF < N_IT)
        def _():
            idx_load(k + NBUF, ib_l)

    def round_body(r, carry):
        for j in range(NI):
            k = r * NI + j
            visit(k, j % NBUF, j)
        return carry

    n_rounds = (N_IT + NI) // NI
    lax.fori_loop(0, n_rounds, round_body, 0)
    if with_deg:
        pltpu.sync_copy(hist_v, deg_hbm.at[wid])
    plsc.subcore_barrier()

    pltpu.sync_copy(agg_s.at[pl.ds(r0, rows_ps)],
                    out_hbm.at[pl.ds(cid * N_PAD + r0, rows_ps)])

  return _edge_scatter


_edge_scatter_l1 = _make_edge_scatter(True)
_edge_scatter_l2 = _make_edge_scatter(False)


def _pre1_body(h_ref, qt_ref, qb_ref, o_ref):
    m = jnp.dot(h_ref[...], qt_ref[...], preferred_element_type=jnp.float32)
    o_ref[...] = jnp.maximum(m + qb_ref[...], 0.0)


def _mid_body(h_ref, a0_ref, a1_ref, degp_ref, w1a_ref, w1b_ref, w1b_b_ref,
              q2t_ref, q2b_ref, h1_ref, m2_ref, deg_ref):
    deg = jnp.sum(degp_ref[...], axis=0)
    hn = (a0_ref[...] + a1_ref[...]) / jnp.maximum(deg, 1.0)[:, None]
    z = jnp.dot(h_ref[...], w1a_ref[...], preferred_element_type=jnp.float32)
    z = z + jnp.dot(hn, w1b_ref[...], preferred_element_type=jnp.float32)
    z = jnp.maximum(z + w1b_b_ref[...], 0.0)
    nrm = jnp.sqrt(jnp.sum(z * z, axis=1, keepdims=True))
    h1 = z / (nrm + 1e-6)
    h1_ref[...] = h1
    m2 = jnp.dot(h1, q2t_ref[...], preferred_element_type=jnp.float32)
    m2_ref[...] = jnp.maximum(m2 + q2b_ref[...], 0.0)
    deg_ref[...] = deg


def _post_body(h1_ref, a0_ref, a1_ref, deg_ref, w2a_ref, w2b_ref, w2b_b_ref,
               gt_ref, gb_ref, gs_ref, o_ref):
    deg = deg_ref[...]
    hn = (a0_ref[...] + a1_ref[...]) / jnp.maximum(deg, 1.0)[:, None]
    z = jnp.dot(h1_ref[...], w2a_ref[...], preferred_element_type=jnp.float32)
    z = z + jnp.dot(hn, w2b_ref[...], preferred_element_type=jnp.float32)
    z = jnp.maximum(z + w2b_b_ref[...], 0.0)
    nrm = jnp.sqrt(jnp.sum(z * z, axis=1, keepdims=True))
    h2 = z / (nrm + 1e-6)
    out = jnp.dot(h2, gt_ref[...], preferred_element_type=jnp.float32)
    o_ref[...] = gs_ref[...] * jnp.maximum(out + gb_ref[...], 0.0)


def _row_spec(width):
    return pl.BlockSpec((BLK, width), lambda i: (i, 0))


def _full_spec(shape):
    return pl.BlockSpec(shape, lambda i: tuple(0 for _ in shape))


_GRID = N_PAD // BLK
_vec_spec = pl.BlockSpec((BLK,), lambda i: (i,))

_pre1 = pl.pallas_call(
    _pre1_body,
    grid=(_GRID,),
    in_specs=[_row_spec(D), _full_spec((D, D)), _full_spec((1, D))],
    out_specs=_row_spec(D),
    out_shape=jax.ShapeDtypeStruct((N, D), jnp.float32),
)

_a1_spec = pl.BlockSpec((BLK, D), lambda i: (i + _GRID, 0))

_mid = pl.pallas_call(
    _mid_body,
    grid=(_GRID,),
    in_specs=[_row_spec(D), _row_spec(D), _a1_spec,
              pl.BlockSpec((NW, BLK), lambda i: (0, i)),
              _full_spec((D, D)), _full_spec((D, D)), _full_spec((1, D)),
              _full_spec((D, D)), _full_spec((1, D))],
    out_specs=[_row_spec(D), _row_spec(D), _vec_spec],
    out_shape=[jax.ShapeDtypeStruct((N, D), jnp.float32),
               jax.ShapeDtypeStruct((N, D), jnp.float32),
               jax.ShapeDtypeStruct((N,), jnp.float32)],
)

_post = pl.pallas_call(
    _post_body,
    grid=(_GRID,),
    in_specs=[_row_spec(D), _row_spec(D), _a1_spec, _vec_spec,
              _full_spec((D, D)), _full_spec((D, D)), _full_spec((1, D)),
              _full_spec((D, D)), _full_spec((1, D)), _full_spec((1, D))],
    out_specs=_row_spec(D),
    out_shape=jax.ShapeDtypeStruct((N, D), jnp.float32),
)


@jax.jit
def kernel(g, h, Q1_w, Q1_b, W1_w, W1_b, Q2_w, Q2_b, W2_w, W2_b, G_w, G_b,
           g_scale):
    gi = g.astype(jnp.int32)
    sd4d = jnp.stack([gi[0].reshape(NW, N_IT, CHUNK),
                      gi[1].reshape(NW, N_IT, CHUNK)], axis=2)

    zeros_nd = jnp.zeros((N_PAD, D), jnp.float32)
    zeros_1d = jnp.zeros((N_PAD,), jnp.float32)

    m1 = _pre1(h, Q1_w.T, Q1_b[None, :])
    agg1, degp = _edge_scatter_l1(sd4d, m1, zeros_nd, zeros_1d)

    h1, m2, deg = _mid(h, agg1, agg1, degp,
                       W1_w[:, :D].T, W1_w[:, D:].T, W1_b[None, :],
                       Q2_w.T, Q2_b[None, :])

    agg2, = _edge_scatter_l2(sd4d, m2, zeros_nd)

    gs = jnp.broadcast_to(g_scale.astype(jnp.float32), (1, D))
    out = _post(h1, agg2, agg2, deg,
                W2_w[:, :D].T, W2_w[:, D:].T, W2_b[None, :],
                G_w.T, G_b[None, :], gs)
    return out
